# baseline (speedup 1.0000x reference)
import functools
import jax
import jax.numpy as jnp
from jax import lax
from jax.experimental import pallas as pl
from jax.experimental.pallas import tpu as pltpu
from jax.experimental.pallas import tpu_sc as plsc

_BF0, _BF1, _BF2, _WF3T, _DATA, _D, _S = 0, 128, 256, 320, 448, 464, 592
_WS0, _WM0, _WG0, _BG0, _B0 = _S, _S + 5, _S + 10, _S + 12, _S + 13
_WS1, _WM1, _WG1, _BG1, _B1 = _S + 23, _S + 73, _S + 123, _S + 143, _S + 144
_WS2, _WM2, _WG2, _BG2, _B2 = _S + 154, _S + 204, _S + 254, _S + 274, _S + 275
_PACK = 880


def _sc_impl(pack, eip, Wf0, Wf1, Wf2):
    mesh = plsc.VectorSubcoreMesh(core_axis_name="c", subcore_axis_name="s")

    @functools.partial(
        pl.kernel, mesh=mesh,
        out_type=jax.ShapeDtypeStruct((16,), jnp.float32),
        compiler_params=pltpu.CompilerParams(
            needs_layout_passes=False, use_tc_tiling_on_sc=False),
        scratch_types=[
            pltpu.VMEM((_PACK,), jnp.float32),
            pltpu.VMEM((192,), jnp.int32),
            pltpu.VMEM((220, 128), jnp.float32),
            pltpu.VMEM((128, 128), jnp.float32),
            pltpu.VMEM((128, 64), jnp.float32),
            pltpu.VMEM((128,), jnp.float32),
            pltpu.VMEM((224,), jnp.float32),
            pltpu.VMEM((128,), jnp.float32),
            pltpu.VMEM((16,), jnp.float32),
            pltpu.SemaphoreType.DMA((5,)),
        ],
    )
    def k(pack_hbm, eip_hbm, Wf0_hbm, Wf1_hbm, Wf2_hbm, out_hbm,
          pack_s, eip_s, Wf0_s, Wf1_s, Wf2_s, agg_s, flat_s, h_s, out_s, sem):
        wid = lax.axis_index("c") * 16 + lax.axis_index("s")

        @pl.when(wid == 0)
        def _():
            hbms = (pack_hbm, eip_hbm, Wf0_hbm, Wf1_hbm, Wf2_hbm)
            vmems = (pack_s, eip_s, Wf0_s, Wf1_s, Wf2_s)
            copies = [pltpu.make_async_copy(h, v, sem.at[i])
                      for i, (h, v) in enumerate(zip(hbms, vmems))]
            for c in copies:
                c.start()
            for c in copies:
                c.wait()

            lanes = lax.broadcasted_iota(jnp.int32, (16,), 0)
            m10 = lanes < 10
            zf = jnp.zeros((16,), jnp.float32)

            blocks = {}

            def blk(b):
                if b not in blocks:
                    blocks[b] = pack_s[pl.ds(16 * b, 16)]
                return blocks[b]

            def bc(off):
                return blk(off // 16).at[
                    jnp.full((16,), off % 16, jnp.int32)].get(
                        mode="promise_in_bounds")

            def sigmoid(v):
                return 1.0 / (1.0 + jnp.exp(-v))

            def leaky(v):
                return jnp.where(v >= 0, v, 0.01 * v)

            def message_pass(p_gd, p_gs, pm, ps, bg_off, b_off):
                for j in range(5):
                    agg_s[pl.ds(16 * j, 16)] = zf
                bgv = bc(bg_off)
                for g in range(6):
                    mv = (lanes + 16 * g) < 90
                    srcv = eip_s[pl.ds(16 * g, 16)]
                    dstv = eip_s[pl.ds(96 + 16 * g, 16)]
                    srcv = jnp.where(mv, srcv, 0)
                    dstv = jnp.where(mv, dstv, 0)
                    pd_e = p_gd.at[dstv].get(mode="promise_in_bounds")
                    ps_e = p_gs.at[srcv].get(mode="promise_in_bounds")
                    gate = sigmoid(pd_e + ps_e + bgv)
                    for j in range(5):
                        pm_e = pm[j].at[srcv].get(mode="promise_in_bounds")
                        plsc.addupdate_scatter(
                            agg_s, [16 * j + dstv], gate * pm_e, mask=mv)
                xnew = []
                for c in range(5):
                    xnew.append(leaky(ps[c] + bc(b_off + c)))
                for c in range(5):
                    aggv = agg_s[pl.ds(16 * c, 16)]
                    xnew.append(leaky(aggv + bc(b_off + 5 + c)))
                return xnew

            xv = jnp.where(m10, blk(_DATA // 16), 0.0)
            p_gd = xv * bc(_WG0)
            p_gs = xv * bc(_WG0 + 1)
            pm = [xv * bc(_WM0 + j) for j in range(5)]
            ps = [xv * bc(_WS0 + j) for j in range(5)]
            x = message_pass(p_gd, p_gs, pm, ps, _BG0, _B0)

            for WsO, WmO, WgO, bgO, bO in ((_WS1, _WM1, _WG1, _BG1, _B1),
                                           (_WS2, _WM2, _WG2, _BG2, _B2)):
                p_gd, p_gs = zf, zf
                pm = [zf] * 5
                ps = [zf] * 5
                for c in range(10):
                    p_gd = p_gd + x[c] * bc(WgO + c)
                    p_gs = p_gs + x[c] * bc(WgO + 10 + c)
                    for j in range(5):
                        pm[j] = pm[j] + x[c] * bc(WmO + 5 * c + j)
                        ps[j] = ps[j] + x[c] * bc(WsO + 5 * c + j)
                x = message_pass(p_gd, p_gs, pm, ps, bgO, bO)

            for c in range(10):
                plsc.store_scatter(flat_s, [lanes * 10 + c], x[c], mask=m10)
            for i in range(8):
                dv = pack_s[pl.ds(_D + 16 * i, 16)]
                plsc.store_scatter(flat_s, [100 + 16 * i + lanes], dv,
                                   mask=(16 * i + lanes) < 120)

            def matvec_loop(n_in, src_ref, w_ref, n_out_regs):
                def body(kk, carry):
                    b = kk // 16
                    block = src_ref[pl.ds(16 * b, 16)]
                    wk = block.at[jnp.full((16,), 0, jnp.int32) +
                                  (kk - 16 * b)].get(
                                      mode="promise_in_bounds")
                    return tuple(
                        carry[r] + wk * w_ref[kk, pl.ds(16 * r, 16)]
                        for r in range(n_out_regs))
                return lax.fori_loop(0, n_in, body, (zf,) * n_out_regs)

            h0 = matvec_loop(220, flat_s, Wf0_s, 8)
            for r in range(8):
                h_s[pl.ds(16 * r, 16)] = leaky(h0[r] + blk(_BF0 // 16 + r))
            h1 = matvec_loop(128, h_s, Wf1_s, 8)
            for r in range(8):
                h_s[pl.ds(16 * r, 16)] = leaky(h1[r] + blk(_BF1 // 16 + r))
            h2 = matvec_loop(128, h_s, Wf2_s, 4)
            for r in range(4):
                h_s[pl.ds(16 * r, 16)] = leaky(h2[r] + blk(_BF2 // 16 + r))

            acc0, acc1 = zf, zf
            for r in range(4):
                h2v = h_s[pl.ds(16 * r, 16)]
                acc0 = acc0 + h2v * blk(_WF3T // 16 + r)
                acc1 = acc1 + h2v * blk(_WF3T // 16 + 4 + r)
            s0 = jnp.cumsum(acc0)[15] + bc(_S + 285)
            s1 = jnp.cumsum(acc1)[15] + bc(_S + 286)
            res = jnp.where(lanes == 0, sigmoid(s0),
                            jnp.where(lanes == 1, sigmoid(s1), 0.0))
            out_s[pl.ds(0, 16)] = res
            pltpu.sync_copy(out_s, out_hbm)

    return k(pack, eip, Wf0, Wf1, Wf2)


def kernel(data, d, edge_index, Ws0, Wm0, Wg0, bg0, b0, Ws1, Wm1, Wg1, bg1, b1,
           Ws2, Wm2, Wg2, bg2, b2, Wf0, bf0, Wf1, bf1, Wf2, bf2, Wf3, bf3):
    z = jnp.zeros
    pack = jnp.concatenate([
        bf0, bf1, bf2, Wf3.T.reshape(-1),
        data.reshape(-1), z((6,), jnp.float32),
        d.reshape(-1), z((8,), jnp.float32),
        Ws0.reshape(-1), Wm0.reshape(-1), Wg0.reshape(-1), bg0, b0,
        Ws1.reshape(-1), Wm1.reshape(-1), Wg1.reshape(-1), bg1, b1,
        Ws2.reshape(-1), Wm2.reshape(-1), Wg2.reshape(-1), bg2, b2,
        bf3, z((1,), jnp.float32),
    ])
    eip = jnp.pad(edge_index.astype(jnp.int32), ((0, 0), (0, 6))).reshape(-1)
    out = _sc_impl(pack, eip, Wf0, Wf1, Wf2)
    return out[0:2]


# speedup vs baseline: 5.3991x; 5.3991x over previous
import jax
import jax.numpy as jnp
from jax import lax
from jax.experimental import pallas as pl


def _sigmoid(x):
    return 1.0 / (1.0 + jnp.exp(-x))


def _leaky(x):
    return jnp.where(x >= 0, x, 0.01 * x)


def _fused_body(dataT_ref, d_ref, ei_ref,
                Ws0_ref, Wm0_ref, Wg0T_ref, bg0_ref, b0_ref,
                Ws1T_ref, Wm1T_ref, Wg1T_ref, bg1_ref, b1_ref,
                Ws2T_ref, Wm2T_ref, Wg2T_ref, bg2_ref, b2_ref,
                Wf0_ref, bf0_ref, Wf1_ref, bf1_ref,
                Wf2T_ref, bf2_ref, Wf3T_ref, bf3_ref,
                out_ref):
    E = 90
    N = 10
    ei = ei_ref[...]
    node_iota = lax.broadcasted_iota(jnp.int32, (N, E), 0)
    ST = (ei[0:1, :] == node_iota).astype(jnp.float32)
    DT = (ei[1:2, :] == node_iota).astype(jnp.float32)

    def edges_and_agg(P, bg, b, dout):
        Pd = lax.dot_general(DT, P[:, 0:1],
                             (((0,), (0,)), ((), ())),
                             preferred_element_type=jnp.float32)
        Ps = lax.dot_general(ST, P[:, 1:2 + dout],
                             (((0,), (0,)), ((), ())),
                             preferred_element_type=jnp.float32)
        gate = _sigmoid(Pd + Ps[:, 0:1] + bg[0, 0])
        msg = gate * Ps[:, 1:]
        agg = lax.dot_general(DT, msg,
                              (((1,), (0,)), ((), ())),
                              preferred_element_type=jnp.float32)
        h = jnp.concatenate([P[:, 2 + dout:2 + 2 * dout], agg], axis=1) + b
        return _leaky(h)

    Wg0T = Wg0T_ref[...]
    Wcat0 = jnp.concatenate([Wg0T[:, 0:1], Wg0T[:, 1:2],
                             Wm0_ref[...], Ws0_ref[...]], axis=1)
    P = lax.dot_general(dataT_ref[...], Wcat0,
                        (((0,), (0,)), ((), ())),
                        preferred_element_type=jnp.float32)
    x = edges_and_agg(P, bg0_ref, b0_ref[...], 5)

    for WsT_ref, WmT_ref, WgT_ref, bg_ref, b_ref in (
            (Ws1T_ref, Wm1T_ref, Wg1T_ref, bg1_ref, b1_ref),
            (Ws2T_ref, Wm2T_ref, Wg2T_ref, bg2_ref, b2_ref)):
        WgT = WgT_ref[...]
        WcatT = jnp.concatenate([WgT[:, 0:N], WgT[:, N:2 * N],
                                 WmT_ref[...], WsT_ref[...]], axis=0)
        P = lax.dot_general(x, WcatT,
                            (((1,), (1,)), ((), ())),
                            preferred_element_type=jnp.float32)
        x = edges_and_agg(P, bg_ref, b_ref[...], 5)

    def row_flatten(a, cols):
        rep = jnp.concatenate([a] * N, axis=1)
        k_iota = lax.broadcasted_iota(jnp.int32, (N, N * cols), 1)
        n_iota = lax.broadcasted_iota(jnp.int32, (N, N * cols), 0)
        mask = (k_iota // cols) == n_iota
        spread = jnp.where(mask, rep, 0.0)
        ones = jnp.ones((1, N), jnp.float32)
        return jnp.dot(ones, spread, preferred_element_type=jnp.float32)

    x_flat = row_flatten(x, 10)
    d_flat = row_flatten(d_ref[...], 12)
    flat = jnp.concatenate([x_flat, d_flat], axis=1)

    h = _leaky(jnp.dot(flat, Wf0_ref[...], preferred_element_type=jnp.float32)
               + bf0_ref[...])
    h = _leaky(jnp.dot(h, Wf1_ref[...], preferred_element_type=jnp.float32)
               + bf1_ref[...])
    h = _leaky(lax.dot_general(h, Wf2T_ref[...],
                               (((1,), (1,)), ((), ())),
                               preferred_element_type=jnp.float32)
               + bf2_ref[...])
    h = _sigmoid(lax.dot_general(h, Wf3T_ref[...],
                                 (((1,), (1,)), ((), ())),
                                 preferred_element_type=jnp.float32)
                 + bf3_ref[...])
    out_ref[...] = h


def kernel(data, d, edge_index, Ws0, Wm0, Wg0, bg0, b0, Ws1, Wm1, Wg1, bg1, b1,
           Ws2, Wm2, Wg2, bg2, b2, Wf0, bf0, Wf1, bf1, Wf2, bf2, Wf3, bf3):
    out = pl.pallas_call(
        _fused_body,
        out_shape=jax.ShapeDtypeStruct((1, 2), jnp.float32),
    )(data.T, d, edge_index.astype(jnp.int32),
      Ws0, Wm0, Wg0.T, bg0.reshape(1, -1), b0.reshape(1, -1),
      Ws1.T, Wm1.T, Wg1.T, bg1.reshape(1, -1), b1.reshape(1, -1),
      Ws2.T, Wm2.T, Wg2.T, bg2.reshape(1, -1), b2.reshape(1, -1),
      Wf0, bf0.reshape(1, -1), Wf1, bf1.reshape(1, -1),
      Wf2.T, bf2.reshape(1, -1), Wf3.T, bf3.reshape(1, -1))
    return out.reshape(2)
